# baseline (speedup 1.0000x reference)
import functools

import jax
import jax.numpy as jnp
from jax import lax
from jax.experimental import pallas as pl
from jax.experimental.pallas import tpu as pltpu
from jax.experimental.pallas import tpu_sc as plsc

NUM_CLASSES = 80
SCORE_THRESH = 0.05
NMS_THRESH = 0.5
TOPK = 100
PRE_NMS = 2000
IMG_W = 1333.0
IMG_H = 800.0
N_BOXES = 20000

C = 4096
CR, CL = 32, 128
OFFS = max(IMG_W, IMG_H) + 1.0
NEG_INF = float("-inf")


def _nms_body(m_ref, s_ref, f_ref, x1_ref, y1_ref, x2_ref, y2_ref, out_ref,
              ox1_s, oy1_s, ox2_s, oy2_s, ar_s, cx1_s, cy1_s, cx2_s, cy2_s,
              sc_s, cl_s):
    M = m_ref[0]
    s = s_ref[...]
    f = f_ref[...]
    row_i = lax.broadcasted_iota(jnp.int32, (CR, CL), 0)
    lane_i = lax.broadcasted_iota(jnp.int32, (CR, CL), 1)
    flat_i = row_i * CL + lane_i

    sbits = lax.bitcast_convert_type(s, jnp.int32)
    valid = (flat_i < M) & (s > 0.0)
    ebits = jnp.where(valid, sbits, -1)

    def bs_body(_, carry):
        lo, hi = carry
        mid = (lo + hi) // 2
        cnt = jnp.sum((ebits >= mid).astype(jnp.int32))
        ge = cnt >= PRE_NMS
        return (jnp.where(ge, mid, lo), jnp.where(ge, hi, mid))

    lo, _ = lax.fori_loop(0, 31, bs_body, (jnp.int32(0), jnp.int32(0x3F800001)))
    tau = lo
    n_gt = jnp.sum((ebits > tau).astype(jnp.int32))
    eq = ebits == tau
    n_eq = jnp.sum(eq.astype(jnp.int32))
    needed = jnp.minimum(PRE_NMS - n_gt, n_eq)

    def bs2_body(_, carry):
        lo2, hi2 = carry
        mid = (lo2 + hi2) // 2
        cnt = jnp.sum((eq & (f <= mid)).astype(jnp.int32))
        ge = cnt >= needed
        return (jnp.where(ge, lo2, mid), jnp.where(ge, mid, hi2))

    _, phi = lax.fori_loop(0, 22, bs2_body,
                           (jnp.int32(-1), jnp.int32(N_BOXES * NUM_CLASSES)))
    keep_eq = eq & (f <= phi) & (needed > 0)
    alive = (ebits > tau) | keep_eq
    work0 = jnp.where(alive, s, NEG_INF)

    cx1 = jnp.clip(x1_ref[...], 0.0, IMG_W)
    cy1 = jnp.clip(y1_ref[...], 0.0, IMG_H)
    cx2 = jnp.clip(x2_ref[...], 0.0, IMG_W)
    cy2 = jnp.clip(y2_ref[...], 0.0, IMG_H)
    cls_i = jnp.where(valid, f % NUM_CLASSES, 0)
    clsf = cls_i.astype(jnp.float32)
    off = clsf * OFFS
    ox1 = cx1 + off
    oy1 = cy1 + off
    ox2 = cx2 + off
    oy2 = cy2 + off
    areas = (ox2 - ox1) * (oy2 - oy1)

    ox1_s[...] = ox1
    oy1_s[...] = oy1
    ox2_s[...] = ox2
    oy2_s[...] = oy2
    ar_s[...] = areas
    cx1_s[...] = cx1
    cy1_s[...] = cy1
    cx2_s[...] = cx2
    cy2_s[...] = cy2
    sc_s[...] = s
    cl_s[...] = clsf

    lane8 = lax.broadcasted_iota(jnp.int32, (1, 8), 1)

    def loop_body(t, work):
        m = jnp.max(work)
        jj = jnp.min(jnp.where(work == m, flat_i, C))
        r = jj // CL
        c = jj % CL
        onehot = lane_i[0:1, :] == c

        def ext(ref):
            return jnp.sum(jnp.where(onehot, ref[pl.ds(r, 1), :], 0.0))

        bx1 = ext(ox1_s)
        by1 = ext(oy1_s)
        bx2 = ext(ox2_s)
        by2 = ext(oy2_s)
        bar = ext(ar_s)
        picked = m > NEG_INF

        ix1 = jnp.maximum(bx1, ox1)
        iy1 = jnp.maximum(by1, oy1)
        ix2 = jnp.minimum(bx2, ox2)
        iy2 = jnp.minimum(by2, oy2)
        iw = jnp.maximum(ix2 - ix1, 0.0)
        ih = jnp.maximum(iy2 - iy1, 0.0)
        inter = iw * ih
        union = jnp.maximum(areas + bar - inter, 1e-6)
        iou = inter / union
        suppress = iou > NMS_THRESH
        work = jnp.where(suppress, NEG_INF, work)

        vx1 = ext(cx1_s)
        vy1 = ext(cy1_s)
        vx2 = ext(cx2_s)
        vy2 = ext(cy2_s)
        vsc = ext(sc_s)
        vcl = ext(cl_s)
        row = (jnp.where(lane8 == 0, vx1, 0.0) + jnp.where(lane8 == 1, vy1, 0.0)
               + jnp.where(lane8 == 2, vx2, 0.0) + jnp.where(lane8 == 3, vy2, 0.0)
               + jnp.where(lane8 == 4, vsc, 0.0) + jnp.where(lane8 == 5, vcl, 0.0))
        pad = jnp.where(lane8 == 5, -1.0, 0.0)
        out_ref[pl.ds(t, 1), :] = jnp.where(picked, row, pad)
        return work

    lax.fori_loop(0, TOPK, loop_body, work0)


def _nms_call(m_arr, cs, cf, bx1, by1, bx2, by2):
    scr = [pltpu.VMEM((CR, CL), jnp.float32)] * 11
    out = pl.pallas_call(
        _nms_body,
        out_shape=jax.ShapeDtypeStruct((TOPK, 8), jnp.float32),
        in_specs=[pl.BlockSpec(memory_space=pltpu.SMEM)]
        + [pl.BlockSpec(memory_space=pltpu.VMEM)] * 6,
        scratch_shapes=scr,
    )(m_arr, cs, cf, bx1, by1, bx2, by2)
    return out[:, :6]


NT = 16
ROWS = N_BOXES // NT
NB = 144
BUCKET_BASE = 3923
BUCKET_SHIFT = 18
TILE_CAP = C + 16
CH = 128


def _sel_body(scores_hbm, boxes_hbm,
              out_s, out_f, out_x1, out_y1, out_x2, out_y2, out_cnt,
              sbuf, csc, cfl, hist, ghb, allh,
              gi0, gi1, gi2, gi3, gx0, gx1, gx2, gx3,
              posb, cbuf, allc, hist_sh, cnt_sh):
    wid = lax.axis_index("s")
    iota = lax.iota(jnp.int32, 16)
    zv = jnp.zeros((16,), jnp.int32)
    ones_i = jnp.ones((16,), jnp.int32)
    th = jnp.float32(SCORE_THRESH)

    pltpu.sync_copy(scores_hbm.at[wid], sbuf)

    def zh(i, _):
        hist[pl.ds(i * 16, 16)] = zv
        return 0
    lax.fori_loop(0, (NB * 16) // 16, zh, 0)

    def zc(i, _):
        cfl[pl.ds(i * 16, 16)] = zv
        return 0
    lax.fori_loop(0, TILE_CAP // 16, zc, 0)

    laneoff = iota * NB - BUCKET_BASE

    def hrow(row, _):
        for cc in range(5):
            v = sbuf[row, pl.ds(cc * 16, 16)]
            m = v > th
            b = lax.shift_right_logical(
                plsc.bitcast(v, jnp.int32), BUCKET_SHIFT) + laneoff
            plsc.addupdate_scatter(hist, [b], ones_i, mask=m)
        return 0
    lax.fori_loop(0, ROWS, hrow, 0)

    for k in range(NB // 16):
        acc = zv
        for l in range(16):
            acc = acc + hist[pl.ds(l * NB + k * 16, 16)]
        ghb[pl.ds(k * 16, 16)] = acc
    pltpu.sync_copy(ghb, hist_sh.at[wid])
    plsc.subcore_barrier()
    pltpu.sync_copy(hist_sh, allh)

    carry = jnp.int32(0)
    bstar = jnp.int32(0)
    bhi = jnp.int32(-1)
    for k in range(NB // 16 - 1, -1, -1):
        g = zv
        for t in range(NT):
            g = g + allh[t, pl.ds(k * 16, 16)]
        sfx = lax.rev(plsc.cumsum(lax.rev(g, (0,))), (0,)) + carry
        idxk = iota + (k * 16)
        bstar = jnp.maximum(bstar, jnp.max(jnp.where(sfx >= PRE_NMS, idxk, -1)))
        bhi = jnp.maximum(bhi, jnp.max(jnp.where(sfx > C, idxk, -1)))
        carry = carry + jnp.sum(g)
    bsel = jnp.maximum(bstar, bhi + 1)
    edge_bits = lax.shift_left(bsel + BUCKET_BASE, BUCKET_SHIFT)

    def srow(row, car):
        offv, idxv = car
        for cc in range(5):
            v = sbuf[row, pl.ds(cc * 16, 16)]
            eb = plsc.bitcast(v, jnp.int32)
            m = (v > th) & (eb >= edge_bits)
            mi = m.astype(jnp.int32)
            pos = offv + plsc.cumsum(mi) - mi
            plsc.store_scatter(csc, [pos], v, mask=m)
            plsc.store_scatter(cfl, [pos], idxv, mask=m)
            offv = offv + plsc.all_reduce_population_count(m)
            idxv = idxv + 16
        return (offv, idxv)

    base_flat = wid * (ROWS * NUM_CLASSES)
    offv, _ = lax.fori_loop(
        0, ROWS, srow, (jnp.zeros((16,), jnp.int32), base_flat + iota))
    off = jnp.max(offv)

    cbuf[...] = jnp.where(iota == 0, off, 0)
    pltpu.sync_copy(cbuf, cnt_sh.at[wid])
    plsc.subcore_barrier()
    pltpu.sync_copy(cnt_sh, allc)
    prefix = jnp.int32(0)
    M = jnp.int32(0)
    for t in range(NT):
        nt_ = jnp.sum(jnp.where(iota == 0, allc[t, pl.ds(0, 16)], 0))
        prefix = prefix + jnp.where(wid > t, nt_, 0)
        M = M + nt_

    @pl.when(wid == 0)
    def _():
        cbuf[...] = jnp.where(iota == 0, M, 0)
        pltpu.sync_copy(cbuf.at[pl.ds(0, 8)], out_cnt)

    nch = (off + CH - 1) // CH

    def chunk(q, _):
        base = q * CH
        for i in range(CH // 16):
            fl = cfl[pl.ds(base + i * 16, 16)]
            d4 = (fl // NUM_CLASSES) * 4
            gi0[pl.ds(i * 16, 16)] = d4
            gi1[pl.ds(i * 16, 16)] = d4 + 1
            gi2[pl.ds(i * 16, 16)] = d4 + 2
            gi3[pl.ds(i * 16, 16)] = d4 + 3
            pv = base + i * 16 + iota
            posb[pl.ds(i * 16, 16)] = jnp.where(pv < off, prefix + pv, C)
        pltpu.sync_copy(boxes_hbm.at[gi0], gx0)
        pltpu.sync_copy(boxes_hbm.at[gi1], gx1)
        pltpu.sync_copy(boxes_hbm.at[gi2], gx2)
        pltpu.sync_copy(boxes_hbm.at[gi3], gx3)
        pltpu.sync_copy(csc.at[pl.ds(base, CH)], out_s.at[posb])
        pltpu.sync_copy(cfl.at[pl.ds(base, CH)], out_f.at[posb])
        pltpu.sync_copy(gx0, out_x1.at[posb])
        pltpu.sync_copy(gx1, out_y1.at[posb])
        pltpu.sync_copy(gx2, out_x2.at[posb])
        pltpu.sync_copy(gx3, out_y2.at[posb])
        return 0
    lax.fori_loop(0, nch, chunk, 0)


def _sel_call(scores, boxes_flat):
    f32, i32 = jnp.float32, jnp.int32
    out_type = [
        jax.ShapeDtypeStruct((C + 8,), f32),
        jax.ShapeDtypeStruct((C + 8,), i32),
        jax.ShapeDtypeStruct((C + 8,), f32),
        jax.ShapeDtypeStruct((C + 8,), f32),
        jax.ShapeDtypeStruct((C + 8,), f32),
        jax.ShapeDtypeStruct((C + 8,), f32),
        jax.ShapeDtypeStruct((8,), i32),
    ]
    scratch = [
        pltpu.VMEM((ROWS, NUM_CLASSES + 1), f32),
        pltpu.VMEM((TILE_CAP,), f32),
        pltpu.VMEM((TILE_CAP,), i32),
        pltpu.VMEM((NB * 16,), i32),
        pltpu.VMEM((NB,), i32),
        pltpu.VMEM((NT, NB), i32),
        pltpu.VMEM((CH,), i32),
        pltpu.VMEM((CH,), i32),
        pltpu.VMEM((CH,), i32),
        pltpu.VMEM((CH,), i32),
        pltpu.VMEM((CH,), f32),
        pltpu.VMEM((CH,), f32),
        pltpu.VMEM((CH,), f32),
        pltpu.VMEM((CH,), f32),
        pltpu.VMEM((CH,), i32),
        pltpu.VMEM((16,), i32),
        pltpu.VMEM((NT, 16), i32),
        pltpu.VMEM_SHARED((NT, NB), i32),
        pltpu.VMEM_SHARED((NT, 16), i32),
    ]
    mesh = plsc.VectorSubcoreMesh(
        core_axis_name="c", subcore_axis_name="s", num_cores=1)
    fn = pl.kernel(_sel_body, out_type=out_type, mesh=mesh,
                   scratch_types=scratch,
                   compiler_params=pltpu.CompilerParams(
                       use_tc_tiling_on_sc=False,
                       needs_layout_passes=False))
    return fn(scores.reshape(NT, ROWS, NUM_CLASSES + 1), boxes_flat)


def kernel(boxes, scores):
    out_s, out_f, ox1, oy1, ox2, oy2, out_cnt = _sel_call(
        scores, boxes.reshape(-1))
    m_arr = out_cnt[:1]
    return _nms_call(m_arr,
                     out_s[:C].reshape(CR, CL), out_f[:C].reshape(CR, CL),
                     ox1[:C].reshape(CR, CL), oy1[:C].reshape(CR, CL),
                     ox2[:C].reshape(CR, CL), oy2[:C].reshape(CR, CL))


# speedup vs baseline: 1.0304x; 1.0304x over previous
import functools

import jax
import jax.numpy as jnp
from jax import lax
from jax.experimental import pallas as pl
from jax.experimental.pallas import tpu as pltpu
from jax.experimental.pallas import tpu_sc as plsc

NUM_CLASSES = 80
SCORE_THRESH = 0.05
NMS_THRESH = 0.5
TOPK = 100
PRE_NMS = 2000
IMG_W = 1333.0
IMG_H = 800.0
N_BOXES = 20000

C = 4096
CR, CL = 32, 128
OFFS = max(IMG_W, IMG_H) + 1.0
NEG_INF = float("-inf")


def _nms_body(m_ref, s_ref, f_ref, x1_ref, y1_ref, x2_ref, y2_ref, out_ref,
              ox1_s, oy1_s, ox2_s, oy2_s, ar_s, cx1_s, cy1_s, cx2_s, cy2_s,
              sc_s, cl_s):
    M = m_ref[0]
    s = s_ref[...]
    f = f_ref[...]
    row_i = lax.broadcasted_iota(jnp.int32, (CR, CL), 0)
    lane_i = lax.broadcasted_iota(jnp.int32, (CR, CL), 1)
    flat_i = row_i * CL + lane_i

    sbits = lax.bitcast_convert_type(s, jnp.int32)
    valid = (flat_i < M) & (s > 0.0)
    ebits = jnp.where(valid, sbits, -1)

    def bs_body(_, carry):
        lo, hi = carry
        mid = (lo + hi) // 2
        cnt = jnp.sum((ebits >= mid).astype(jnp.int32))
        ge = cnt >= PRE_NMS
        return (jnp.where(ge, mid, lo), jnp.where(ge, hi, mid))

    lo, _ = lax.fori_loop(0, 31, bs_body, (jnp.int32(0), jnp.int32(0x3F800001)))
    tau = lo
    n_gt = jnp.sum((ebits > tau).astype(jnp.int32))
    eq = ebits == tau
    n_eq = jnp.sum(eq.astype(jnp.int32))
    needed = jnp.minimum(PRE_NMS - n_gt, n_eq)

    def bs2_body(_, carry):
        lo2, hi2 = carry
        mid = (lo2 + hi2) // 2
        cnt = jnp.sum((eq & (f <= mid)).astype(jnp.int32))
        ge = cnt >= needed
        return (jnp.where(ge, lo2, mid), jnp.where(ge, mid, hi2))

    _, phi = lax.fori_loop(0, 22, bs2_body,
                           (jnp.int32(-1), jnp.int32(N_BOXES * NUM_CLASSES)))
    keep_eq = eq & (f <= phi) & (needed > 0)
    alive = (ebits > tau) | keep_eq
    work0 = jnp.where(alive, s, NEG_INF)

    cx1 = jnp.clip(x1_ref[...], 0.0, IMG_W)
    cy1 = jnp.clip(y1_ref[...], 0.0, IMG_H)
    cx2 = jnp.clip(x2_ref[...], 0.0, IMG_W)
    cy2 = jnp.clip(y2_ref[...], 0.0, IMG_H)
    cls_i = jnp.where(valid, f % NUM_CLASSES, 0)
    clsf = cls_i.astype(jnp.float32)
    off = clsf * OFFS
    ox1 = cx1 + off
    oy1 = cy1 + off
    ox2 = cx2 + off
    oy2 = cy2 + off
    areas = (ox2 - ox1) * (oy2 - oy1)

    ox1_s[...] = ox1
    oy1_s[...] = oy1
    ox2_s[...] = ox2
    oy2_s[...] = oy2
    ar_s[...] = areas
    cx1_s[...] = cx1
    cy1_s[...] = cy1
    cx2_s[...] = cx2
    cy2_s[...] = cy2
    sc_s[...] = s
    cl_s[...] = clsf

    lane8 = lax.broadcasted_iota(jnp.int32, (1, 8), 1)

    def loop_body(t, work):
        m = jnp.max(work)
        jj = jnp.min(jnp.where(work == m, flat_i, C))
        r = jj // CL
        c = jj % CL
        onehot = lane_i[0:1, :] == c

        def ext(ref):
            return jnp.sum(jnp.where(onehot, ref[pl.ds(r, 1), :], 0.0))

        bx1 = ext(ox1_s)
        by1 = ext(oy1_s)
        bx2 = ext(ox2_s)
        by2 = ext(oy2_s)
        bar = ext(ar_s)
        picked = m > NEG_INF

        ix1 = jnp.maximum(bx1, ox1)
        iy1 = jnp.maximum(by1, oy1)
        ix2 = jnp.minimum(bx2, ox2)
        iy2 = jnp.minimum(by2, oy2)
        iw = jnp.maximum(ix2 - ix1, 0.0)
        ih = jnp.maximum(iy2 - iy1, 0.0)
        inter = iw * ih
        union = jnp.maximum(areas + bar - inter, 1e-6)
        iou = inter / union
        suppress = iou > NMS_THRESH
        work = jnp.where(suppress, NEG_INF, work)

        vx1 = ext(cx1_s)
        vy1 = ext(cy1_s)
        vx2 = ext(cx2_s)
        vy2 = ext(cy2_s)
        vsc = ext(sc_s)
        vcl = ext(cl_s)
        row = (jnp.where(lane8 == 0, vx1, 0.0) + jnp.where(lane8 == 1, vy1, 0.0)
               + jnp.where(lane8 == 2, vx2, 0.0) + jnp.where(lane8 == 3, vy2, 0.0)
               + jnp.where(lane8 == 4, vsc, 0.0) + jnp.where(lane8 == 5, vcl, 0.0))
        pad = jnp.where(lane8 == 5, -1.0, 0.0)
        out_ref[pl.ds(t, 1), :] = jnp.where(picked, row, pad)
        return work

    lax.fori_loop(0, TOPK, loop_body, work0)


def _nms_call(m_arr, cs, cf, bx1, by1, bx2, by2):
    scr = [pltpu.VMEM((CR, CL), jnp.float32)] * 11
    out = pl.pallas_call(
        _nms_body,
        out_shape=jax.ShapeDtypeStruct((TOPK, 8), jnp.float32),
        in_specs=[pl.BlockSpec(memory_space=pltpu.SMEM)]
        + [pl.BlockSpec(memory_space=pltpu.VMEM)] * 6,
        scratch_shapes=scr,
    )(m_arr, cs, cf, bx1, by1, bx2, by2)
    return out[:, :6]


NT = 16
ROWS = N_BOXES // NT
NB = 144
BUCKET_BASE = 3923
BUCKET_SHIFT = 18
TILE_CAP = C + 16
CH = 128


def _sel_body(scores_hbm, boxes_hbm,
              out_s, out_f, out_x1, out_y1, out_x2, out_y2, out_cnt,
              sbuf, csc, cfl, hist, ghb, allh,
              gi0, gi1, gi2, gi3, gx0, gx1, gx2, gx3,
              posb, cbuf, allc, hist_sh, cnt_sh):
    wid = lax.axis_index("s")
    iota = lax.iota(jnp.int32, 16)
    zv = jnp.zeros((16,), jnp.int32)
    ones_i = jnp.ones((16,), jnp.int32)
    th = jnp.float32(SCORE_THRESH)

    pltpu.sync_copy(scores_hbm.at[wid], sbuf)

    def zh(i, _):
        hist[pl.ds(i * 16, 16)] = zv
        return 0
    lax.fori_loop(0, (NB * 16) // 16, zh, 0)

    def zc(i, _):
        cfl[pl.ds(i * 16, 16)] = zv
        return 0
    lax.fori_loop(0, TILE_CAP // 16, zc, 0)

    laneoff = iota * NB - BUCKET_BASE

    def hrow(row, _):
        for cc in range(5):
            v = sbuf[row, pl.ds(cc * 16, 16)]
            m = v > th
            b = lax.shift_right_logical(
                plsc.bitcast(v, jnp.int32), BUCKET_SHIFT) + laneoff
            plsc.addupdate_scatter(hist, [b], ones_i, mask=m)
        return 0
    lax.fori_loop(0, ROWS, hrow, 0)

    for k in range(NB // 16):
        acc = zv
        for l in range(16):
            acc = acc + hist[pl.ds(l * NB + k * 16, 16)]
        ghb[pl.ds(k * 16, 16)] = acc
    pltpu.sync_copy(ghb, hist_sh.at[wid])
    plsc.subcore_barrier()
    pltpu.sync_copy(hist_sh, allh)

    carry = jnp.int32(0)
    bstar = jnp.int32(0)
    bhi = jnp.int32(-1)
    for k in range(NB // 16 - 1, -1, -1):
        g = zv
        for t in range(NT):
            g = g + allh[t, pl.ds(k * 16, 16)]
        sfx = lax.rev(plsc.cumsum(lax.rev(g, (0,))), (0,)) + carry
        idxk = iota + (k * 16)
        bstar = jnp.maximum(bstar, jnp.max(jnp.where(sfx >= PRE_NMS, idxk, -1)))
        bhi = jnp.maximum(bhi, jnp.max(jnp.where(sfx > C, idxk, -1)))
        carry = carry + jnp.sum(g)
    bsel = jnp.maximum(bstar, bhi + 1)
    edge_bits = lax.shift_left(bsel + BUCKET_BASE, BUCKET_SHIFT)

    th_bits = jnp.int32(0x3D4CCCCD + 1)
    sel_bits = jnp.maximum(edge_bits, th_bits)

    def srow(row, car):
        offv, rbase = car
        ms = []
        for cc in range(5):
            v = sbuf[row, pl.ds(cc * 16, 16)]
            ms.append(plsc.bitcast(v, jnp.int32) >= sel_bits)
        anyv = ((ms[0] | ms[1]) | (ms[2] | ms[3])) | ms[4]
        nhit = jnp.max(plsc.all_reduce_population_count(anyv))

        def slow(offv):
            o = offv
            for cc in range(5):
                v = sbuf[row, pl.ds(cc * 16, 16)]
                m = ms[cc]
                mi = m.astype(jnp.int32)
                pos = o + plsc.cumsum(mi) - mi
                plsc.store_scatter(csc, [pos], v, mask=m)
                plsc.store_scatter(cfl, [pos], rbase + (cc * 16) + iota, mask=m)
                o = o + plsc.all_reduce_population_count(m)
            return o

        offv = lax.cond(nhit > 0, slow, lambda o: o, offv)
        return (offv, rbase + NUM_CLASSES)

    base_flat = wid * (ROWS * NUM_CLASSES)
    offv, _ = lax.fori_loop(
        0, ROWS, srow, (jnp.zeros((16,), jnp.int32), base_flat + iota))
    off = jnp.max(offv)

    cbuf[...] = jnp.where(iota == 0, off, 0)
    pltpu.sync_copy(cbuf, cnt_sh.at[wid])
    plsc.subcore_barrier()
    pltpu.sync_copy(cnt_sh, allc)
    prefix = jnp.int32(0)
    M = jnp.int32(0)
    for t in range(NT):
        nt_ = jnp.sum(jnp.where(iota == 0, allc[t, pl.ds(0, 16)], 0))
        prefix = prefix + jnp.where(wid > t, nt_, 0)
        M = M + nt_

    @pl.when(wid == 0)
    def _():
        cbuf[...] = jnp.where(iota == 0, M, 0)
        pltpu.sync_copy(cbuf.at[pl.ds(0, 8)], out_cnt)

    nch = (off + CH - 1) // CH

    def chunk(q, _):
        base = q * CH
        for i in range(CH // 16):
            fl = cfl[pl.ds(base + i * 16, 16)]
            d4 = (fl // NUM_CLASSES) * 4
            gi0[pl.ds(i * 16, 16)] = d4
            gi1[pl.ds(i * 16, 16)] = d4 + 1
            gi2[pl.ds(i * 16, 16)] = d4 + 2
            gi3[pl.ds(i * 16, 16)] = d4 + 3
            pv = base + i * 16 + iota
            posb[pl.ds(i * 16, 16)] = jnp.where(pv < off, prefix + pv, C)
        pltpu.sync_copy(boxes_hbm.at[gi0], gx0)
        pltpu.sync_copy(boxes_hbm.at[gi1], gx1)
        pltpu.sync_copy(boxes_hbm.at[gi2], gx2)
        pltpu.sync_copy(boxes_hbm.at[gi3], gx3)
        pltpu.sync_copy(csc.at[pl.ds(base, CH)], out_s.at[posb])
        pltpu.sync_copy(cfl.at[pl.ds(base, CH)], out_f.at[posb])
        pltpu.sync_copy(gx0, out_x1.at[posb])
        pltpu.sync_copy(gx1, out_y1.at[posb])
        pltpu.sync_copy(gx2, out_x2.at[posb])
        pltpu.sync_copy(gx3, out_y2.at[posb])
        return 0
    lax.fori_loop(0, nch, chunk, 0)


def _sel_call(scores, boxes_flat):
    f32, i32 = jnp.float32, jnp.int32
    out_type = [
        jax.ShapeDtypeStruct((C + 8,), f32),
        jax.ShapeDtypeStruct((C + 8,), i32),
        jax.ShapeDtypeStruct((C + 8,), f32),
        jax.ShapeDtypeStruct((C + 8,), f32),
        jax.ShapeDtypeStruct((C + 8,), f32),
        jax.ShapeDtypeStruct((C + 8,), f32),
        jax.ShapeDtypeStruct((8,), i32),
    ]
    scratch = [
        pltpu.VMEM((ROWS, NUM_CLASSES + 1), f32),
        pltpu.VMEM((TILE_CAP,), f32),
        pltpu.VMEM((TILE_CAP,), i32),
        pltpu.VMEM((NB * 16,), i32),
        pltpu.VMEM((NB,), i32),
        pltpu.VMEM((NT, NB), i32),
        pltpu.VMEM((CH,), i32),
        pltpu.VMEM((CH,), i32),
        pltpu.VMEM((CH,), i32),
        pltpu.VMEM((CH,), i32),
        pltpu.VMEM((CH,), f32),
        pltpu.VMEM((CH,), f32),
        pltpu.VMEM((CH,), f32),
        pltpu.VMEM((CH,), f32),
        pltpu.VMEM((CH,), i32),
        pltpu.VMEM((16,), i32),
        pltpu.VMEM((NT, 16), i32),
        pltpu.VMEM_SHARED((NT, NB), i32),
        pltpu.VMEM_SHARED((NT, 16), i32),
    ]
    mesh = plsc.VectorSubcoreMesh(
        core_axis_name="c", subcore_axis_name="s", num_cores=1)
    fn = pl.kernel(_sel_body, out_type=out_type, mesh=mesh,
                   scratch_types=scratch,
                   compiler_params=pltpu.CompilerParams(
                       use_tc_tiling_on_sc=False,
                       needs_layout_passes=False))
    return fn(scores.reshape(NT, ROWS, NUM_CLASSES + 1), boxes_flat)


def kernel(boxes, scores):
    out_s, out_f, ox1, oy1, ox2, oy2, out_cnt = _sel_call(
        scores, boxes.reshape(-1))
    m_arr = out_cnt[:1]
    return _nms_call(m_arr,
                     out_s[:C].reshape(CR, CL), out_f[:C].reshape(CR, CL),
                     ox1[:C].reshape(CR, CL), oy1[:C].reshape(CR, CL),
                     ox2[:C].reshape(CR, CL), oy2[:C].reshape(CR, CL))


# speedup vs baseline: 1.0734x; 1.0418x over previous
import functools

import jax
import jax.numpy as jnp
from jax import lax
from jax.experimental import pallas as pl
from jax.experimental.pallas import tpu as pltpu
from jax.experimental.pallas import tpu_sc as plsc

NUM_CLASSES = 80
SCORE_THRESH = 0.05
NMS_THRESH = 0.5
TOPK = 100
PRE_NMS = 2000
IMG_W = 1333.0
IMG_H = 800.0
N_BOXES = 20000

C = 4096
CR, CL = 32, 128
OFFS = max(IMG_W, IMG_H) + 1.0
NEG_INF = float("-inf")


def _nms_body(m_ref, s_ref, f_ref, x1_ref, y1_ref, x2_ref, y2_ref, out_ref,
              ox1_s, oy1_s, ox2_s, oy2_s, ar_s, cx1_s, cy1_s, cx2_s, cy2_s,
              sc_s, cl_s):
    M = m_ref[0]
    s = s_ref[...]
    f = f_ref[...]
    row_i = lax.broadcasted_iota(jnp.int32, (CR, CL), 0)
    lane_i = lax.broadcasted_iota(jnp.int32, (CR, CL), 1)
    flat_i = row_i * CL + lane_i

    sbits = lax.bitcast_convert_type(s, jnp.int32)
    valid = (flat_i < M) & (s > 0.0)
    ebits = jnp.where(valid, sbits, -1)

    def bs_body(_, carry):
        lo, hi = carry
        mid = (lo + hi) // 2
        cnt = jnp.sum((ebits >= mid).astype(jnp.int32))
        ge = cnt >= PRE_NMS
        return (jnp.where(ge, mid, lo), jnp.where(ge, hi, mid))

    lo, _ = lax.fori_loop(0, 31, bs_body, (jnp.int32(0), jnp.int32(0x3F800001)))
    tau = lo
    n_gt = jnp.sum((ebits > tau).astype(jnp.int32))
    eq = ebits == tau
    n_eq = jnp.sum(eq.astype(jnp.int32))
    needed = jnp.minimum(PRE_NMS - n_gt, n_eq)

    def bs2_body(_, carry):
        lo2, hi2 = carry
        mid = (lo2 + hi2) // 2
        cnt = jnp.sum((eq & (f <= mid)).astype(jnp.int32))
        ge = cnt >= needed
        return (jnp.where(ge, lo2, mid), jnp.where(ge, mid, hi2))

    _, phi = lax.fori_loop(0, 22, bs2_body,
                           (jnp.int32(-1), jnp.int32(N_BOXES * NUM_CLASSES)))
    keep_eq = eq & (f <= phi) & (needed > 0)
    alive = (ebits > tau) | keep_eq
    work0 = jnp.where(alive, s, NEG_INF)

    cx1 = jnp.clip(x1_ref[...], 0.0, IMG_W)
    cy1 = jnp.clip(y1_ref[...], 0.0, IMG_H)
    cx2 = jnp.clip(x2_ref[...], 0.0, IMG_W)
    cy2 = jnp.clip(y2_ref[...], 0.0, IMG_H)
    cls_i = jnp.where(valid, f % NUM_CLASSES, 0)
    clsf = cls_i.astype(jnp.float32)
    off = clsf * OFFS
    ox1 = cx1 + off
    oy1 = cy1 + off
    ox2 = cx2 + off
    oy2 = cy2 + off
    areas = (ox2 - ox1) * (oy2 - oy1)

    ox1_s[...] = ox1
    oy1_s[...] = oy1
    ox2_s[...] = ox2
    oy2_s[...] = oy2
    ar_s[...] = areas
    cx1_s[...] = cx1
    cy1_s[...] = cy1
    cx2_s[...] = cx2
    cy2_s[...] = cy2
    sc_s[...] = s
    cl_s[...] = clsf

    lane8 = lax.broadcasted_iota(jnp.int32, (1, 8), 1)

    def loop_body(t, work):
        m = jnp.max(work)
        jj = jnp.min(jnp.where(work == m, flat_i, C))
        r = jj // CL
        c = jj % CL
        onehot = lane_i[0:1, :] == c

        def ext(ref):
            return jnp.sum(jnp.where(onehot, ref[pl.ds(r, 1), :], 0.0))

        bx1 = ext(ox1_s)
        by1 = ext(oy1_s)
        bx2 = ext(ox2_s)
        by2 = ext(oy2_s)
        bar = ext(ar_s)
        picked = m > NEG_INF

        ix1 = jnp.maximum(bx1, ox1)
        iy1 = jnp.maximum(by1, oy1)
        ix2 = jnp.minimum(bx2, ox2)
        iy2 = jnp.minimum(by2, oy2)
        iw = jnp.maximum(ix2 - ix1, 0.0)
        ih = jnp.maximum(iy2 - iy1, 0.0)
        inter = iw * ih
        union = jnp.maximum(areas + bar - inter, 1e-6)
        iou = inter / union
        suppress = iou > NMS_THRESH
        work = jnp.where(suppress, NEG_INF, work)

        vx1 = ext(cx1_s)
        vy1 = ext(cy1_s)
        vx2 = ext(cx2_s)
        vy2 = ext(cy2_s)
        vsc = ext(sc_s)
        vcl = ext(cl_s)
        row = (jnp.where(lane8 == 0, vx1, 0.0) + jnp.where(lane8 == 1, vy1, 0.0)
               + jnp.where(lane8 == 2, vx2, 0.0) + jnp.where(lane8 == 3, vy2, 0.0)
               + jnp.where(lane8 == 4, vsc, 0.0) + jnp.where(lane8 == 5, vcl, 0.0))
        pad = jnp.where(lane8 == 5, -1.0, 0.0)
        out_ref[pl.ds(t, 1), :] = jnp.where(picked, row, pad)
        return work

    lax.fori_loop(0, TOPK, loop_body, work0)


def _nms_call(m_arr, cs, cf, bx1, by1, bx2, by2):
    scr = [pltpu.VMEM((CR, CL), jnp.float32)] * 11
    out = pl.pallas_call(
        _nms_body,
        out_shape=jax.ShapeDtypeStruct((TOPK, 8), jnp.float32),
        in_specs=[pl.BlockSpec(memory_space=pltpu.SMEM)]
        + [pl.BlockSpec(memory_space=pltpu.VMEM)] * 6,
        scratch_shapes=scr,
    )(m_arr, cs, cf, bx1, by1, bx2, by2)
    return out[:, :6]


NT = 16
ROWS = N_BOXES // NT
NB = 144
BUCKET_BASE = 3923
BUCKET_SHIFT = 18
TILE_CAP = C + 16
CH = 128


def _sel_body(scores_hbm, boxes_hbm,
              out_s, out_f, out_x1, out_y1, out_x2, out_y2, out_cnt,
              sbuf, csc, cfl, hist, ghb, allh,
              gi0, gi1, gi2, gi3, gx0, gx1, gx2, gx3,
              posb, cbuf, allc, hist_sh, cnt_sh, dsem):
    wid = lax.axis_index("s")
    iota = lax.iota(jnp.int32, 16)
    zv = jnp.zeros((16,), jnp.int32)
    ones_i = jnp.ones((16,), jnp.int32)
    th = jnp.float32(SCORE_THRESH)

    pltpu.sync_copy(scores_hbm.at[wid], sbuf)

    def zh(i, _):
        hist[pl.ds(i * 16, 16)] = zv
        return 0
    lax.fori_loop(0, (NB * 16) // 16, zh, 0)

    def zc(i, _):
        cfl[pl.ds(i * 16, 16)] = zv
        return 0
    lax.fori_loop(0, TILE_CAP // 16, zc, 0)

    laneoff = iota * NB - BUCKET_BASE

    def hrow(row, _):
        for cc in range(5):
            v = sbuf[row, pl.ds(cc * 16, 16)]
            m = v > th
            b = lax.shift_right_logical(
                plsc.bitcast(v, jnp.int32), BUCKET_SHIFT) + laneoff
            plsc.addupdate_scatter(hist, [b], ones_i, mask=m)
        return 0
    lax.fori_loop(0, ROWS, hrow, 0)

    for k in range(NB // 16):
        acc = zv
        for l in range(16):
            acc = acc + hist[pl.ds(l * NB + k * 16, 16)]
        ghb[pl.ds(k * 16, 16)] = acc
    pltpu.sync_copy(ghb, hist_sh.at[wid])
    plsc.subcore_barrier()
    pltpu.sync_copy(hist_sh, allh)

    carry = jnp.int32(0)
    bstar = jnp.int32(0)
    bhi = jnp.int32(-1)
    for k in range(NB // 16 - 1, -1, -1):
        g = zv
        for t in range(NT):
            g = g + allh[t, pl.ds(k * 16, 16)]
        sfx = lax.rev(plsc.cumsum(lax.rev(g, (0,))), (0,)) + carry
        idxk = iota + (k * 16)
        bstar = jnp.maximum(bstar, jnp.max(jnp.where(sfx >= PRE_NMS, idxk, -1)))
        bhi = jnp.maximum(bhi, jnp.max(jnp.where(sfx > C, idxk, -1)))
        carry = carry + jnp.sum(g)
    bsel = jnp.maximum(bstar, bhi + 1)
    edge_bits = lax.shift_left(bsel + BUCKET_BASE, BUCKET_SHIFT)

    th_bits = jnp.int32(0x3D4CCCCD + 1)
    sel_bits = jnp.maximum(edge_bits, th_bits)

    def srow(row, car):
        offv, rbase = car
        ms = []
        for cc in range(5):
            v = sbuf[row, pl.ds(cc * 16, 16)]
            ms.append(plsc.bitcast(v, jnp.int32) >= sel_bits)
        anyv = ((ms[0] | ms[1]) | (ms[2] | ms[3])) | ms[4]
        nhit = jnp.max(plsc.all_reduce_population_count(anyv))

        def slow(offv):
            o = offv
            for cc in range(5):
                v = sbuf[row, pl.ds(cc * 16, 16)]
                m = ms[cc]
                mi = m.astype(jnp.int32)
                pos = o + plsc.cumsum(mi) - mi
                plsc.store_scatter(csc, [pos], v, mask=m)
                plsc.store_scatter(cfl, [pos], rbase + (cc * 16) + iota, mask=m)
                o = o + plsc.all_reduce_population_count(m)
            return o

        offv = lax.cond(nhit > 0, slow, lambda o: o, offv)
        return (offv, rbase + NUM_CLASSES)

    base_flat = wid * (ROWS * NUM_CLASSES)
    offv, _ = lax.fori_loop(
        0, ROWS, srow, (jnp.zeros((16,), jnp.int32), jnp.int32(base_flat)))
    off = jnp.max(offv)

    cbuf[...] = jnp.where(iota == 0, off, 0)
    pltpu.sync_copy(cbuf, cnt_sh.at[wid])
    plsc.subcore_barrier()
    pltpu.sync_copy(cnt_sh, allc)
    prefix = jnp.int32(0)
    M = jnp.int32(0)
    for t in range(NT):
        nt_ = jnp.sum(jnp.where(iota == 0, allc[t, pl.ds(0, 16)], 0))
        prefix = prefix + jnp.where(wid > t, nt_, 0)
        M = M + nt_

    @pl.when(wid == 0)
    def _():
        cbuf[...] = jnp.where(iota == 0, M, 0)
        pltpu.sync_copy(cbuf.at[pl.ds(0, 8)], out_cnt)

    nch = (off + CH - 1) // CH

    def chunk(q, _):
        base = q * CH
        for i in range(CH // 16):
            fl = cfl[pl.ds(base + i * 16, 16)]
            d4 = (fl // NUM_CLASSES) * 4
            gi0[pl.ds(i * 16, 16)] = d4
            gi1[pl.ds(i * 16, 16)] = d4 + 1
            gi2[pl.ds(i * 16, 16)] = d4 + 2
            gi3[pl.ds(i * 16, 16)] = d4 + 3
            pv = base + i * 16 + iota
            posb[pl.ds(i * 16, 16)] = jnp.where(pv < off, prefix + pv, C)
        hg = [pltpu.make_async_copy(boxes_hbm.at[gi0], gx0, dsem),
              pltpu.make_async_copy(boxes_hbm.at[gi1], gx1, dsem),
              pltpu.make_async_copy(boxes_hbm.at[gi2], gx2, dsem),
              pltpu.make_async_copy(boxes_hbm.at[gi3], gx3, dsem)]
        for h in hg:
            h.start()
        for h in hg:
            h.wait()
        hs = [pltpu.make_async_copy(csc.at[pl.ds(base, CH)],
                                    out_s.at[posb], dsem),
              pltpu.make_async_copy(cfl.at[pl.ds(base, CH)],
                                    out_f.at[posb], dsem),
              pltpu.make_async_copy(gx0, out_x1.at[posb], dsem),
              pltpu.make_async_copy(gx1, out_y1.at[posb], dsem),
              pltpu.make_async_copy(gx2, out_x2.at[posb], dsem),
              pltpu.make_async_copy(gx3, out_y2.at[posb], dsem)]
        for h in hs:
            h.start()
        for h in hs:
            h.wait()
        return 0
    lax.fori_loop(0, nch, chunk, 0)


def _sel_call(scores, boxes_flat):
    f32, i32 = jnp.float32, jnp.int32
    out_type = [
        jax.ShapeDtypeStruct((C + 8,), f32),
        jax.ShapeDtypeStruct((C + 8,), i32),
        jax.ShapeDtypeStruct((C + 8,), f32),
        jax.ShapeDtypeStruct((C + 8,), f32),
        jax.ShapeDtypeStruct((C + 8,), f32),
        jax.ShapeDtypeStruct((C + 8,), f32),
        jax.ShapeDtypeStruct((8,), i32),
    ]
    scratch = [
        pltpu.VMEM((ROWS, NUM_CLASSES + 1), f32),
        pltpu.VMEM((TILE_CAP,), f32),
        pltpu.VMEM((TILE_CAP,), i32),
        pltpu.VMEM((NB * 16,), i32),
        pltpu.VMEM((NB,), i32),
        pltpu.VMEM((NT, NB), i32),
        pltpu.VMEM((CH,), i32),
        pltpu.VMEM((CH,), i32),
        pltpu.VMEM((CH,), i32),
        pltpu.VMEM((CH,), i32),
        pltpu.VMEM((CH,), f32),
        pltpu.VMEM((CH,), f32),
        pltpu.VMEM((CH,), f32),
        pltpu.VMEM((CH,), f32),
        pltpu.VMEM((CH,), i32),
        pltpu.VMEM((16,), i32),
        pltpu.VMEM((NT, 16), i32),
        pltpu.VMEM_SHARED((NT, NB), i32),
        pltpu.VMEM_SHARED((NT, 16), i32),
        pltpu.SemaphoreType.DMA,
    ]
    mesh = plsc.VectorSubcoreMesh(
        core_axis_name="c", subcore_axis_name="s", num_cores=1)
    fn = pl.kernel(_sel_body, out_type=out_type, mesh=mesh,
                   scratch_types=scratch,
                   compiler_params=pltpu.CompilerParams(
                       use_tc_tiling_on_sc=False,
                       needs_layout_passes=False))
    return fn(scores.reshape(NT, ROWS, NUM_CLASSES + 1), boxes_flat)


def kernel(boxes, scores):
    out_s, out_f, ox1, oy1, ox2, oy2, out_cnt = _sel_call(
        scores, boxes.reshape(-1))
    m_arr = out_cnt[:1]
    return _nms_call(m_arr,
                     out_s[:C].reshape(CR, CL), out_f[:C].reshape(CR, CL),
                     ox1[:C].reshape(CR, CL), oy1[:C].reshape(CR, CL),
                     ox2[:C].reshape(CR, CL), oy2[:C].reshape(CR, CL))
